# baseline (speedup 1.0000x reference)
import functools

import jax
import jax.numpy as jnp
from jax import lax
from jax.experimental import pallas as pl
from jax.experimental.pallas import tpu as pltpu
from jax.experimental.pallas import tpu_sc as plsc

N = 10000
E = 160000
H = 256
HH = 128
H2 = 512
L = 5
G = 64
NAF = 9
NBF = 3
AV = 128
BV = 8

NTILES = 16
K = 64
ET = 10240
E_PAD = NTILES * ET
NCH = ET // K
AGG_ROWS = 10112
DUMMY = N
ROWS_PER_TILE = AGG_ROWS // NTILES
ZCHUNKS = AGG_ROWS // K
ZPT = -(-ZCHUNKS // NTILES)
NBC = 128

_PREC = lax.Precision.HIGHEST
_f32 = jnp.float32


def _edge_half(sd_r, b_r, h_r, et_r, out_r, idxb, bcb, src_k, dst_k, bc_k,
               hbufs, ebufs, aggs, isems, bsems, gsems, esems, tid):
    base = tid * ET

    zbuf = hbufs[0]

    @pl.loop(0, K)
    def _zrow(r):
        for c in range(HH // 16):
            zbuf[r, pl.ds(c * 16, 16)] = jnp.zeros((16,), _f32)

    zhi = jnp.minimum((tid + 1) * ZPT, ZCHUNKS)

    @pl.loop(tid * ZPT, zhi)
    def _zchunk(z):
        pltpu.sync_copy(zbuf, aggs.at[pl.ds(z * K, K)])

    plsc.subcore_barrier()

    def _i_desc(j, b):
        return pltpu.make_async_copy(sd_r.at[pl.ds(base + j * K, K)],
                                     idxb[b], isems[b])

    def _b_desc(j, b):
        return pltpu.make_async_copy(b_r.at[pl.ds(base + j * K, K)],
                                     bcb[b], bsems[b])

    def _unpack(b):
        for c in range(K // 16):
            s = pl.ds(c * 16, 16)
            v = idxb[b][s]
            src_k[b][s] = v & 0xFFFF
            dst_k[b][s] = lax.shift_right_logical(v, 16)
            bc_k[b][s] = bcb[b][s]

    def _g_desc(b):
        return pltpu.make_async_copy(h_r.at[src_k[b]], hbufs[b], gsems[b])

    def _e_desc(b):
        return pltpu.make_async_copy(et_r.at[bc_k[b]], ebufs[b], esems[b])

    def _step(chunk, b):
        nb = 1 - b

        @pl.when(chunk + 1 < NCH)
        def _():
            _i_desc(chunk + 1, nb).wait()
            _b_desc(chunk + 1, nb).wait()
            _unpack(nb)
            _g_desc(nb).start()
            _e_desc(nb).start()

        @pl.when(chunk + 2 < NCH)
        def _():
            _i_desc(chunk + 2, b).start()
            _b_desc(chunk + 2, b).start()

        _g_desc(b).wait()
        _e_desc(b).wait()
        hbuf, ebuf = hbufs[b], ebufs[b]

        @pl.loop(0, K, unroll=8)
        def _row(r):
            for c in range(HH // 16):
                s = pl.ds(c * 16, 16)
                hbuf[r, s] = jnp.maximum(hbuf[r, s] + ebuf[r, s], 0.0)

        pltpu.sync_copy(hbuf, aggs.at[dst_k[b]], add=True)

    _i_desc(0, 0).start()
    _b_desc(0, 0).start()
    _i_desc(1, 1).start()
    _b_desc(1, 1).start()
    _i_desc(0, 0).wait()
    _b_desc(0, 0).wait()
    _unpack(0)
    _g_desc(0).start()
    _e_desc(0).start()

    @pl.loop(0, NCH, step=2)
    def _chunk(j):
        _step(j, 0)
        _step(j + 1, 1)

    plsc.subcore_barrier()
    sl = pl.ds(tid * ROWS_PER_TILE, ROWS_PER_TILE)
    pltpu.sync_copy(aggs.at[sl], out_r.at[sl])


def _sc_edge_body(sd_r, b_r, hs_r, et_a, et_b, out_r,
                  ib0, ib1, bb0, bb1, sk0, sk1, dk0, dk1, bk0, bk1,
                  hbuf0, hbuf1, ebuf0, ebuf1, aggs,
                  isem0, isem1, bsem0, bsem1, gsem0, gsem1, esem0, esem1):
    cid = lax.axis_index("c")
    tid = lax.axis_index("s")
    idxb = (ib0, ib1)
    bcb = (bb0, bb1)
    hbufs = (hbuf0, hbuf1)
    ebufs = (ebuf0, ebuf1)
    src_k = (sk0, sk1)
    dst_k = (dk0, dk1)
    bc_k = (bk0, bk1)
    isems = (isem0, isem1)
    bsems = (bsem0, bsem1)
    gsems = (gsem0, gsem1)
    esems = (esem0, esem1)

    @pl.when(cid == 0)
    def _():
        _edge_half(sd_r, b_r, hs_r.at[0], et_a, out_r.at[0], idxb, bcb,
                   src_k, dst_k, bc_k, hbufs, ebufs, aggs, isems, bsems,
                   gsems, esems, tid)

    @pl.when(cid == 1)
    def _():
        _edge_half(sd_r, b_r, hs_r.at[1], et_b, out_r.at[1], idxb, bcb,
                   src_k, dst_k, bc_k, hbufs, ebufs, aggs, isems, bsems,
                   gsems, esems, tid)


@functools.cache
def _build_sc_edge():
    return pl.kernel(
        _sc_edge_body,
        out_type=jax.ShapeDtypeStruct((2, AGG_ROWS, HH), _f32),
        mesh=plsc.VectorSubcoreMesh(core_axis_name="c", subcore_axis_name="s",
                                    num_cores=2, num_subcores=NTILES),
        scratch_types=[
            pltpu.VMEM((K,), jnp.int32),
            pltpu.VMEM((K,), jnp.int32),
            pltpu.VMEM((K,), jnp.int32),
            pltpu.VMEM((K,), jnp.int32),
            pltpu.VMEM((K,), jnp.int32),
            pltpu.VMEM((K,), jnp.int32),
            pltpu.VMEM((K,), jnp.int32),
            pltpu.VMEM((K,), jnp.int32),
            pltpu.VMEM((K,), jnp.int32),
            pltpu.VMEM((K,), jnp.int32),
            pltpu.VMEM((K, HH), _f32),
            pltpu.VMEM((K, HH), _f32),
            pltpu.VMEM((K, HH), _f32),
            pltpu.VMEM((K, HH), _f32),
            pltpu.VMEM_SHARED((AGG_ROWS, HH), _f32),
            pltpu.SemaphoreType.DMA,
            pltpu.SemaphoreType.DMA,
            pltpu.SemaphoreType.DMA,
            pltpu.SemaphoreType.DMA,
            pltpu.SemaphoreType.DMA,
            pltpu.SemaphoreType.DMA,
            pltpu.SemaphoreType.DMA,
            pltpu.SemaphoreType.DMA,
        ],
    )


def _sc_edge(sd_pad, bc_pad, hs, et_a, et_b):
    return _build_sc_edge()(sd_pad, bc_pad, hs, et_a, et_b)


BN_A = 2000


def _atom_body(xr, tabr, o):
    xb = xr[...]
    iot = lax.broadcasted_iota(jnp.int32, (BN_A, NAF * AV), 1)
    oh = jnp.zeros((BN_A, NAF * AV), _f32)
    for i in range(NAF):
        oh = oh + (xb[:, i:i + 1] == iot).astype(_f32)
    hb = jnp.dot(oh, tabr[...], precision=_PREC, preferred_element_type=_f32)
    o[0] = hb[:, :HH]
    o[1] = hb[:, HH:]


def _atom_enc(flat_x, atab):
    return pl.pallas_call(
        _atom_body,
        grid=(N // BN_A,),
        in_specs=[
            pl.BlockSpec((BN_A, NAF), lambda i: (i, 0)),
            pl.BlockSpec((NAF * AV, H), lambda i: (0, 0)),
        ],
        out_specs=pl.BlockSpec((2, BN_A, HH), lambda i: (0, i, 0)),
        out_shape=jax.ShapeDtypeStruct((2, N, HH), _f32),
    )(flat_x, atab)


def _etab_body(digr, tabr, oa, ob):
    dig = digr[...]
    iot = lax.broadcasted_iota(jnp.int32, (NBC, NBF * BV), 1)
    oh = jnp.zeros((NBC, NBF * BV), _f32)
    for i in range(NBF):
        oh = oh + ((dig[:, i:i + 1] + i * BV) == iot).astype(_f32)
    v = jnp.dot(oh, tabr[...], precision=_PREC, preferred_element_type=_f32)
    oa[...] = v[:, :HH]
    ob[...] = v[:, HH:]


def _etab(dig, btab):
    return pl.pallas_call(
        _etab_body,
        out_shape=[jax.ShapeDtypeStruct((NBC, HH), _f32)] * 2,
    )(dig, btab)


CB1 = 128


def _addx_body(hs_r, agg_r, xs):
    xs[0] = hs_r[0] + agg_r[0]


def _addx(hs, aggs):
    return pl.pallas_call(
        _addx_body,
        grid=(2,),
        in_specs=[
            pl.BlockSpec((1, N, HH), lambda i: (i, 0, 0)),
            pl.BlockSpec((1, N, HH), lambda i: (i, 0, 0)),
        ],
        out_specs=pl.BlockSpec((1, N, HH), lambda i: (i, 0, 0)),
        out_shape=jax.ShapeDtypeStruct((2, N, HH), _f32),
    )(hs, aggs)


def _mlp1_body(xs_r, w1r, g1r, be1r, zn, acc):
    i = pl.program_id(1)
    z = jnp.dot(xs_r[0], w1r[...], precision=_PREC,
                preferred_element_type=_f32)

    @pl.when(i == 0)
    def _():
        acc[...] = z

    @pl.when(i == 1)
    def _():
        zf = acc[...] + z
        m = jnp.mean(zf, axis=0, keepdims=True)
        v = jnp.mean((zf - m) ** 2, axis=0, keepdims=True)
        zf = (zf - m) * lax.rsqrt(v + 1e-5) * g1r[...] + be1r[...]
        zn[...] = jnp.maximum(zf, 0.0)


def _mlp2_body(zn_r, w2r, out, acc):
    k = pl.program_id(1)
    u = jnp.dot(zn_r[...], w2r[...], precision=_PREC,
                preferred_element_type=_f32)

    @pl.when(k == 0)
    def _():
        acc[...] = u

    @pl.when(k > 0)
    def _():
        acc[...] += u

    @pl.when(k == H2 // CB1 - 1)
    def _():
        out[0] = acc[...]


def _bnres_body(last, u_r, gnr, bnr, hs_r, out):
    uf = u_r[0]
    m = jnp.mean(uf, axis=0, keepdims=True)
    v = jnp.mean((uf - m) ** 2, axis=0, keepdims=True)
    uf = (uf - m) * lax.rsqrt(v + 1e-5) * gnr[...] + bnr[...]
    if not last:
        uf = jnp.maximum(uf, 0.0)
    out[0] = uf + hs_r[0]


def _mlp(last, hs, aggs, w1, g1, be1, w2, gn, bn):
    xs = _addx(hs, aggs)
    zn = pl.pallas_call(
        _mlp1_body,
        grid=(H2 // CB1, 2),
        in_specs=[
            pl.BlockSpec((1, N, HH), lambda j, i: (i, 0, 0)),
            pl.BlockSpec((HH, CB1), lambda j, i: (i, j)),
            pl.BlockSpec((1, CB1), lambda j, i: (0, j)),
            pl.BlockSpec((1, CB1), lambda j, i: (0, j)),
        ],
        out_specs=pl.BlockSpec((N, CB1), lambda j, i: (0, j)),
        out_shape=jax.ShapeDtypeStruct((N, H2), _f32),
        scratch_shapes=[pltpu.VMEM((N, CB1), _f32)],
    )(xs, w1, g1, be1)
    us = pl.pallas_call(
        _mlp2_body,
        grid=(2, H2 // CB1),
        in_specs=[
            pl.BlockSpec((N, CB1), lambda c, k: (0, k)),
            pl.BlockSpec((CB1, HH), lambda c, k: (k, c)),
        ],
        out_specs=pl.BlockSpec((1, N, HH), lambda c, k: (c, 0, 0)),
        out_shape=jax.ShapeDtypeStruct((2, N, HH), _f32),
        scratch_shapes=[pltpu.VMEM((N, HH), _f32)],
    )(zn, w2)
    return pl.pallas_call(
        functools.partial(_bnres_body, last),
        grid=(2,),
        in_specs=[
            pl.BlockSpec((1, N, HH), lambda c: (c, 0, 0)),
            pl.BlockSpec((1, HH), lambda c: (0, c)),
            pl.BlockSpec((1, HH), lambda c: (0, c)),
            pl.BlockSpec((1, N, HH), lambda c: (c, 0, 0)),
        ],
        out_specs=pl.BlockSpec((1, N, HH), lambda c: (c, 0, 0)),
        out_shape=jax.ShapeDtypeStruct((2, N, HH), _f32),
    )(us, gn, bn, hs)


def _readout_body(hs_r, br, wor, bor, out):
    hs = hs_r[...]
    hcat = jnp.concatenate([hs[0], hs[1]], axis=1)
    bi = br[...]
    oh = (bi == lax.broadcasted_iota(jnp.int32, (N, G), 1)).astype(_f32)
    sums = lax.dot_general(oh, hcat, (((0,), (0,)), ((), ())),
                           precision=_PREC, preferred_element_type=_f32)
    counts = jnp.sum(oh, axis=0)
    pooled = sums / jnp.clip(counts, 1.0)[:, None]
    logit = jnp.dot(pooled, wor[...], precision=_PREC,
                    preferred_element_type=_f32) + bor[...]
    out[...] = jax.nn.sigmoid(logit)


def _readout(hs, bidx2d, wo, bo2d):
    return pl.pallas_call(
        _readout_body,
        out_shape=jax.ShapeDtypeStruct((G, 1), _f32),
    )(hs, bidx2d, wo, bo2d)


def kernel(x, edge_index, edge_feats, batch_idx, atom_emb, bond_emb,
           W1, b1, g1, be1, W2, b2, gn, bn, Wo, bo):
    x = x.astype(jnp.int32)
    flat_x = x + (jnp.arange(NAF, dtype=jnp.int32) * AV)[None, :]
    ef = edge_feats.astype(jnp.int32)
    bc = ef[:, 0] * 25 + ef[:, 1] * 5 + ef[:, 2]
    src_pad = jnp.pad(edge_index[0].astype(jnp.int32), (0, E_PAD - E))
    dst_pad = jnp.pad(edge_index[1].astype(jnp.int32), (0, E_PAD - E),
                      constant_values=DUMMY)
    sd_pad = src_pad | (dst_pad << 16)
    bc_pad = jnp.pad(bc, (0, E_PAD - E))
    atab = atom_emb.reshape(NAF * AV, H)
    btab = bond_emb.reshape(NBF * BV, H)
    codes = jnp.arange(NBC, dtype=jnp.int32)
    dig = jnp.stack([codes // 25, (codes // 5) % 5, codes % 5], axis=1)

    hs = _atom_enc(flat_x, atab)
    et_a, et_b = _etab(dig, btab)
    for l in range(L):
        aggs = _sc_edge(sd_pad, bc_pad, hs, et_a, et_b)
        hs = _mlp(l == L - 1, hs, aggs,
                  W1[l], g1[l][None], be1[l][None],
                  W2[l], gn[l][None], bn[l][None])
    return _readout(hs, batch_idx.astype(jnp.int32)[:, None], Wo, bo[None])


# speedup vs baseline: 1.0016x; 1.0016x over previous
import functools

import jax
import jax.numpy as jnp
from jax import lax
from jax.experimental import pallas as pl
from jax.experimental.pallas import tpu as pltpu
from jax.experimental.pallas import tpu_sc as plsc

N = 10000
E = 160000
H = 256
HH = 128
H2 = 512
L = 5
G = 64
NAF = 9
NBF = 3
AV = 128
BV = 8

NTILES = 16
K = 64
ET = 10240
E_PAD = NTILES * ET
NCH = ET // K
AGG_ROWS = 10112
DUMMY = N
ROWS_PER_TILE = AGG_ROWS // NTILES
ZCHUNKS = AGG_ROWS // K
ZPT = -(-ZCHUNKS // NTILES)
NBC = 128

_PREC = lax.Precision.HIGHEST
_f32 = jnp.float32


def _edge_half(sd_r, b_r, h_r, et_r, out_r, idxb, bcb, src_k, dst_k, bc_k,
               hbufs, ebufs, aggs, isems, bsems, gsems, esems, tid):
    base = tid * ET

    zbuf = hbufs[0]

    @pl.loop(0, K)
    def _zrow(r):
        for c in range(HH // 16):
            zbuf[r, pl.ds(c * 16, 16)] = jnp.zeros((16,), _f32)

    zhi = jnp.minimum((tid + 1) * ZPT, ZCHUNKS)

    @pl.loop(tid * ZPT, zhi)
    def _zchunk(z):
        pltpu.sync_copy(zbuf, aggs.at[pl.ds(z * K, K)])

    plsc.subcore_barrier()

    def _i_desc(j, b):
        return pltpu.make_async_copy(sd_r.at[pl.ds(base + j * K, K)],
                                     idxb[b], isems[b])

    def _b_desc(j, b):
        return pltpu.make_async_copy(b_r.at[pl.ds(base + j * K, K)],
                                     bcb[b], bsems[b])

    def _unpack(b):
        for c in range(K // 16):
            s = pl.ds(c * 16, 16)
            v = idxb[b][s]
            src_k[b][s] = v & 0xFFFF
            dst_k[b][s] = lax.shift_right_logical(v, 16)
            bc_k[b][s] = bcb[b][s]

    def _g_desc(b):
        return pltpu.make_async_copy(h_r.at[src_k[b]], hbufs[b], gsems[b])

    def _e_desc(b):
        return pltpu.make_async_copy(et_r.at[bc_k[b]], ebufs[b], esems[b])

    def _step(chunk, b):
        nb = 1 - b

        @pl.when(chunk + 1 < NCH)
        def _():
            _i_desc(chunk + 1, nb).wait()
            _b_desc(chunk + 1, nb).wait()
            _unpack(nb)
            _g_desc(nb).start()
            _e_desc(nb).start()

        @pl.when(chunk + 2 < NCH)
        def _():
            _i_desc(chunk + 2, b).start()
            _b_desc(chunk + 2, b).start()

        _g_desc(b).wait()
        _e_desc(b).wait()
        hbuf, ebuf = hbufs[b], ebufs[b]

        @pl.loop(0, K, unroll=4)
        def _row(r):
            for c in range(HH // 16):
                s = pl.ds(c * 16, 16)
                hbuf[r, s] = jnp.maximum(hbuf[r, s] + ebuf[r, s], 0.0)

        pltpu.sync_copy(hbuf, aggs.at[dst_k[b]], add=True)

    _i_desc(0, 0).start()
    _b_desc(0, 0).start()
    _i_desc(1, 1).start()
    _b_desc(1, 1).start()
    _i_desc(0, 0).wait()
    _b_desc(0, 0).wait()
    _unpack(0)
    _g_desc(0).start()
    _e_desc(0).start()

    @pl.loop(0, NCH, step=2)
    def _chunk(j):
        _step(j, 0)
        _step(j + 1, 1)

    plsc.subcore_barrier()
    sl = pl.ds(tid * ROWS_PER_TILE, ROWS_PER_TILE)
    pltpu.sync_copy(aggs.at[sl], out_r.at[sl])


def _sc_edge_body(sd_r, b_r, hs_r, et_a, et_b, out_r,
                  ib0, ib1, bb0, bb1, sk0, sk1, dk0, dk1, bk0, bk1,
                  hbuf0, hbuf1, ebuf0, ebuf1, aggs,
                  isem0, isem1, bsem0, bsem1, gsem0, gsem1, esem0, esem1):
    cid = lax.axis_index("c")
    tid = lax.axis_index("s")
    idxb = (ib0, ib1)
    bcb = (bb0, bb1)
    hbufs = (hbuf0, hbuf1)
    ebufs = (ebuf0, ebuf1)
    src_k = (sk0, sk1)
    dst_k = (dk0, dk1)
    bc_k = (bk0, bk1)
    isems = (isem0, isem1)
    bsems = (bsem0, bsem1)
    gsems = (gsem0, gsem1)
    esems = (esem0, esem1)

    @pl.when(cid == 0)
    def _():
        _edge_half(sd_r, b_r, hs_r.at[0], et_a, out_r.at[0], idxb, bcb,
                   src_k, dst_k, bc_k, hbufs, ebufs, aggs, isems, bsems,
                   gsems, esems, tid)

    @pl.when(cid == 1)
    def _():
        _edge_half(sd_r, b_r, hs_r.at[1], et_b, out_r.at[1], idxb, bcb,
                   src_k, dst_k, bc_k, hbufs, ebufs, aggs, isems, bsems,
                   gsems, esems, tid)


@functools.cache
def _build_sc_edge():
    return pl.kernel(
        _sc_edge_body,
        out_type=jax.ShapeDtypeStruct((2, AGG_ROWS, HH), _f32),
        mesh=plsc.VectorSubcoreMesh(core_axis_name="c", subcore_axis_name="s",
                                    num_cores=2, num_subcores=NTILES),
        scratch_types=[
            pltpu.VMEM((K,), jnp.int32),
            pltpu.VMEM((K,), jnp.int32),
            pltpu.VMEM((K,), jnp.int32),
            pltpu.VMEM((K,), jnp.int32),
            pltpu.VMEM((K,), jnp.int32),
            pltpu.VMEM((K,), jnp.int32),
            pltpu.VMEM((K,), jnp.int32),
            pltpu.VMEM((K,), jnp.int32),
            pltpu.VMEM((K,), jnp.int32),
            pltpu.VMEM((K,), jnp.int32),
            pltpu.VMEM((K, HH), _f32),
            pltpu.VMEM((K, HH), _f32),
            pltpu.VMEM((K, HH), _f32),
            pltpu.VMEM((K, HH), _f32),
            pltpu.VMEM_SHARED((AGG_ROWS, HH), _f32),
            pltpu.SemaphoreType.DMA,
            pltpu.SemaphoreType.DMA,
            pltpu.SemaphoreType.DMA,
            pltpu.SemaphoreType.DMA,
            pltpu.SemaphoreType.DMA,
            pltpu.SemaphoreType.DMA,
            pltpu.SemaphoreType.DMA,
            pltpu.SemaphoreType.DMA,
        ],
    )


def _sc_edge(sd_pad, bc_pad, hs, et_a, et_b):
    return _build_sc_edge()(sd_pad, bc_pad, hs, et_a, et_b)


BN_A = 2000


def _atom_body(xr, tabr, o):
    xb = xr[...]
    iot = lax.broadcasted_iota(jnp.int32, (BN_A, NAF * AV), 1)
    oh = jnp.zeros((BN_A, NAF * AV), _f32)
    for i in range(NAF):
        oh = oh + (xb[:, i:i + 1] == iot).astype(_f32)
    hb = jnp.dot(oh, tabr[...], precision=_PREC, preferred_element_type=_f32)
    o[0] = hb[:, :HH]
    o[1] = hb[:, HH:]


def _atom_enc(flat_x, atab):
    return pl.pallas_call(
        _atom_body,
        grid=(N // BN_A,),
        in_specs=[
            pl.BlockSpec((BN_A, NAF), lambda i: (i, 0)),
            pl.BlockSpec((NAF * AV, H), lambda i: (0, 0)),
        ],
        out_specs=pl.BlockSpec((2, BN_A, HH), lambda i: (0, i, 0)),
        out_shape=jax.ShapeDtypeStruct((2, N, HH), _f32),
    )(flat_x, atab)


def _etab_body(digr, tabr, oa, ob):
    dig = digr[...]
    iot = lax.broadcasted_iota(jnp.int32, (NBC, NBF * BV), 1)
    oh = jnp.zeros((NBC, NBF * BV), _f32)
    for i in range(NBF):
        oh = oh + ((dig[:, i:i + 1] + i * BV) == iot).astype(_f32)
    v = jnp.dot(oh, tabr[...], precision=_PREC, preferred_element_type=_f32)
    oa[...] = v[:, :HH]
    ob[...] = v[:, HH:]


def _etab(dig, btab):
    return pl.pallas_call(
        _etab_body,
        out_shape=[jax.ShapeDtypeStruct((NBC, HH), _f32)] * 2,
    )(dig, btab)


CB1 = 128


def _addx_body(hs_r, agg_r, xs):
    xs[0] = hs_r[0] + agg_r[0]


def _addx(hs, aggs):
    return pl.pallas_call(
        _addx_body,
        grid=(2,),
        in_specs=[
            pl.BlockSpec((1, N, HH), lambda i: (i, 0, 0)),
            pl.BlockSpec((1, N, HH), lambda i: (i, 0, 0)),
        ],
        out_specs=pl.BlockSpec((1, N, HH), lambda i: (i, 0, 0)),
        out_shape=jax.ShapeDtypeStruct((2, N, HH), _f32),
    )(hs, aggs)


def _mlp1_body(xs_r, w1r, g1r, be1r, zn, acc):
    i = pl.program_id(1)
    z = jnp.dot(xs_r[0], w1r[...], precision=_PREC,
                preferred_element_type=_f32)

    @pl.when(i == 0)
    def _():
        acc[...] = z

    @pl.when(i == 1)
    def _():
        zf = acc[...] + z
        m = jnp.mean(zf, axis=0, keepdims=True)
        v = jnp.mean((zf - m) ** 2, axis=0, keepdims=True)
        zf = (zf - m) * lax.rsqrt(v + 1e-5) * g1r[...] + be1r[...]
        zn[...] = jnp.maximum(zf, 0.0)


def _mlp2_body(zn_r, w2r, out, acc):
    k = pl.program_id(1)
    u = jnp.dot(zn_r[...], w2r[...], precision=_PREC,
                preferred_element_type=_f32)

    @pl.when(k == 0)
    def _():
        acc[...] = u

    @pl.when(k > 0)
    def _():
        acc[...] += u

    @pl.when(k == H2 // CB1 - 1)
    def _():
        out[0] = acc[...]


def _bnres_body(last, u_r, gnr, bnr, hs_r, out):
    uf = u_r[0]
    m = jnp.mean(uf, axis=0, keepdims=True)
    v = jnp.mean((uf - m) ** 2, axis=0, keepdims=True)
    uf = (uf - m) * lax.rsqrt(v + 1e-5) * gnr[...] + bnr[...]
    if not last:
        uf = jnp.maximum(uf, 0.0)
    out[0] = uf + hs_r[0]


def _mlp(last, hs, aggs, w1, g1, be1, w2, gn, bn):
    xs = _addx(hs, aggs)
    zn = pl.pallas_call(
        _mlp1_body,
        grid=(H2 // CB1, 2),
        in_specs=[
            pl.BlockSpec((1, N, HH), lambda j, i: (i, 0, 0)),
            pl.BlockSpec((HH, CB1), lambda j, i: (i, j)),
            pl.BlockSpec((1, CB1), lambda j, i: (0, j)),
            pl.BlockSpec((1, CB1), lambda j, i: (0, j)),
        ],
        out_specs=pl.BlockSpec((N, CB1), lambda j, i: (0, j)),
        out_shape=jax.ShapeDtypeStruct((N, H2), _f32),
        scratch_shapes=[pltpu.VMEM((N, CB1), _f32)],
    )(xs, w1, g1, be1)
    us = pl.pallas_call(
        _mlp2_body,
        grid=(2, H2 // CB1),
        in_specs=[
            pl.BlockSpec((N, CB1), lambda c, k: (0, k)),
            pl.BlockSpec((CB1, HH), lambda c, k: (k, c)),
        ],
        out_specs=pl.BlockSpec((1, N, HH), lambda c, k: (c, 0, 0)),
        out_shape=jax.ShapeDtypeStruct((2, N, HH), _f32),
        scratch_shapes=[pltpu.VMEM((N, HH), _f32)],
    )(zn, w2)
    return pl.pallas_call(
        functools.partial(_bnres_body, last),
        grid=(2,),
        in_specs=[
            pl.BlockSpec((1, N, HH), lambda c: (c, 0, 0)),
            pl.BlockSpec((1, HH), lambda c: (0, c)),
            pl.BlockSpec((1, HH), lambda c: (0, c)),
            pl.BlockSpec((1, N, HH), lambda c: (c, 0, 0)),
        ],
        out_specs=pl.BlockSpec((1, N, HH), lambda c: (c, 0, 0)),
        out_shape=jax.ShapeDtypeStruct((2, N, HH), _f32),
    )(us, gn, bn, hs)


def _readout_body(hs_r, br, wor, bor, out):
    hs = hs_r[...]
    hcat = jnp.concatenate([hs[0], hs[1]], axis=1)
    bi = br[...]
    oh = (bi == lax.broadcasted_iota(jnp.int32, (N, G), 1)).astype(_f32)
    sums = lax.dot_general(oh, hcat, (((0,), (0,)), ((), ())),
                           precision=_PREC, preferred_element_type=_f32)
    counts = jnp.sum(oh, axis=0)
    pooled = sums / jnp.clip(counts, 1.0)[:, None]
    logit = jnp.dot(pooled, wor[...], precision=_PREC,
                    preferred_element_type=_f32) + bor[...]
    out[...] = jax.nn.sigmoid(logit)


def _readout(hs, bidx2d, wo, bo2d):
    return pl.pallas_call(
        _readout_body,
        out_shape=jax.ShapeDtypeStruct((G, 1), _f32),
    )(hs, bidx2d, wo, bo2d)


def kernel(x, edge_index, edge_feats, batch_idx, atom_emb, bond_emb,
           W1, b1, g1, be1, W2, b2, gn, bn, Wo, bo):
    x = x.astype(jnp.int32)
    flat_x = x + (jnp.arange(NAF, dtype=jnp.int32) * AV)[None, :]
    ef = edge_feats.astype(jnp.int32)
    bc = ef[:, 0] * 25 + ef[:, 1] * 5 + ef[:, 2]
    src_pad = jnp.pad(edge_index[0].astype(jnp.int32), (0, E_PAD - E))
    dst_pad = jnp.pad(edge_index[1].astype(jnp.int32), (0, E_PAD - E),
                      constant_values=DUMMY)
    sd_pad = src_pad | (dst_pad << 16)
    bc_pad = jnp.pad(bc, (0, E_PAD - E))
    atab = atom_emb.reshape(NAF * AV, H)
    btab = bond_emb.reshape(NBF * BV, H)
    codes = jnp.arange(NBC, dtype=jnp.int32)
    dig = jnp.stack([codes // 25, (codes // 5) % 5, codes % 5], axis=1)

    hs = _atom_enc(flat_x, atab)
    et_a, et_b = _etab(dig, btab)
    for l in range(L):
        aggs = _sc_edge(sd_pad, bc_pad, hs, et_a, et_b)
        hs = _mlp(l == L - 1, hs, aggs,
                  W1[l], g1[l][None], be1[l][None],
                  W2[l], gn[l][None], bn[l][None])
    return _readout(hs, batch_idx.astype(jnp.int32)[:, None], Wo, bo[None])
